# baseline (speedup 1.0000x reference)
import functools

import jax
import jax.numpy as jnp
from jax import lax
from jax.experimental import pallas as pl
from jax.experimental.pallas import tpu as pltpu
from jax.experimental.pallas import tpu_sc as plsc

N = 10000
NPAD = 10240
E = 320000
D = 128
NCLS = 40

NC = 2
NS = 16
CHUNK = 128
NCH_TOT = 2560
EPAD = NCH_TOT * CHUNK
RPT = NPAD // NS
STG = 40

_MESH = plsc.VectorSubcoreMesh(core_axis_name="c", subcore_axis_name="s")


@functools.partial(
    pl.kernel,
    out_type=jax.ShapeDtypeStruct((NC, NPAD), jnp.float32),
    mesh=_MESH,
    scratch_types=[
        pltpu.VMEM((NCH_TOT // (NC * NS), CHUNK), jnp.int32),
        pltpu.VMEM((CHUNK,), jnp.float32),
        pltpu.VMEM_SHARED((NPAD,), jnp.float32),
    ],
)
def _sc_degree(dst2d, ones1, zeros1, degp, dstv, ones_v, acc):
    c = lax.axis_index("c")
    s = lax.axis_index("s")
    n_ch = NCH_TOT // (NC * NS)

    pltpu.sync_copy(ones1, ones_v)
    pltpu.sync_copy(zeros1.at[pl.ds(s * RPT, RPT)], acc.at[pl.ds(s * RPT, RPT)])
    pltpu.sync_copy(dst2d.at[pl.ds((c * NS + s) * n_ch, n_ch)], dstv)
    plsc.subcore_barrier()

    def body(j, _):
        pltpu.sync_copy(ones_v, acc.at[dstv.at[j]], add=True)
        return 0

    lax.fori_loop(0, n_ch, body, 0)
    plsc.subcore_barrier()
    pltpu.sync_copy(acc.at[pl.ds(s * RPT, RPT)],
                    degp.at[c, pl.ds(s * RPT, RPT)])


def _gather_scatter_pipeline(table, srcv, dstv, rows, acc, gsems, ssems, n_st):
    def start_g(j):
        b = j % 2
        pltpu.async_copy(table.at[srcv.at[j]], rows.at[b], gsems.at[b])

    def wait_g(j):
        b = j % 2
        pltpu.make_async_copy(table.at[srcv.at[j]], rows.at[b],
                              gsems.at[b]).wait()

    def start_s(j):
        b = j % 2
        pltpu.async_copy(rows.at[b], acc.at[dstv.at[j]], ssems.at[b], add=True)

    def wait_s(j):
        b = j % 2
        pltpu.make_async_copy(rows.at[b], acc.at[dstv.at[j]],
                              ssems.at[b]).wait()

    start_g(0)

    def body(j, _):
        @pl.when(j >= 1)
        def _():
            wait_s(j - 1)

        @pl.when(j + 1 < n_st)
        def _():
            start_g(j + 1)

        wait_g(j)
        start_s(j)
        return 0

    lax.fori_loop(0, n_st, body, 0)
    wait_s(n_st - 1)


@functools.partial(
    pl.kernel,
    out_type=jax.ShapeDtypeStruct((NC, NPAD, D), jnp.float32),
    mesh=_MESH,
    scratch_types=[
        pltpu.VMEM((STG, CHUNK), jnp.int32),
        pltpu.VMEM((STG, CHUNK), jnp.int32),
        pltpu.VMEM((2, CHUNK, D), jnp.float32),
        pltpu.VMEM_SHARED((NPAD, D), jnp.float32),
        pltpu.SemaphoreType.DMA((2,)),
        pltpu.SemaphoreType.DMA((2,)),
    ],
)
def _sc_aggregate(y, src2d, dst2d, zeros128, agg,
                  srcv, dstv, rows, acc, gsems, ssems):
    c = lax.axis_index("c")
    s = lax.axis_index("s")
    n_ch = NCH_TOT // (NC * NS)

    pltpu.sync_copy(zeros128.at[pl.ds(s * RPT, RPT)], acc.at[pl.ds(s * RPT, RPT)])
    plsc.subcore_barrier()

    def stage(t, _):
        pltpu.sync_copy(src2d.at[pl.ds((c * NS + s) * n_ch + t * STG, STG)], srcv)
        pltpu.sync_copy(dst2d.at[pl.ds((c * NS + s) * n_ch + t * STG, STG)], dstv)
        _gather_scatter_pipeline(y, srcv, dstv, rows, acc, gsems, ssems, STG)
        return 0

    lax.fori_loop(0, n_ch // STG, stage, 0)
    plsc.subcore_barrier()
    pltpu.sync_copy(acc.at[pl.ds(s * RPT, RPT)],
                    agg.at[c, pl.ds(s * RPT, RPT)])


BR = 1024


def _inv_sqrt_deg(degp_blk):
    deg = degp_blk[0, :] + degp_blk[1, :]
    return lax.rsqrt(jnp.maximum(deg, 1.0))[:, None]


def _tc_encode_body(text_ref, vis_ref, wt_ref, bt_ref, wv_ref, bv_ref,
                    degp_ref, w0a_ref, w0b_ref, wht_ref, bht_ref,
                    whv_ref, bhv_ref, y_ref, lut_ref, luv_ref):
    h_t = jax.nn.relu(
        jnp.dot(text_ref[...], wt_ref[...], preferred_element_type=jnp.float32)
        + bt_ref[...])
    h_v = jax.nn.relu(
        jnp.dot(vis_ref[...], wv_ref[...], preferred_element_type=jnp.float32)
        + bv_ref[...])
    inv = _inv_sqrt_deg(degp_ref[...])
    y_ref[...] = (
        jnp.dot(h_t, w0a_ref[...], preferred_element_type=jnp.float32)
        + jnp.dot(h_v, w0b_ref[...], preferred_element_type=jnp.float32)) * inv
    lut_ref[...] = jnp.dot(h_t, wht_ref[...],
                           preferred_element_type=jnp.float32) + bht_ref[...]
    luv_ref[...] = jnp.dot(h_v, whv_ref[...],
                           preferred_element_type=jnp.float32) + bhv_ref[...]


_tc_encode = pl.pallas_call(
    _tc_encode_body,
    grid=(-(-N // BR),),
    in_specs=[
        pl.BlockSpec((BR, D), lambda i: (i, 0)),
        pl.BlockSpec((BR, D), lambda i: (i, 0)),
        pl.BlockSpec((D, D), lambda i: (0, 0)),
        pl.BlockSpec((1, D), lambda i: (0, 0)),
        pl.BlockSpec((D, D), lambda i: (0, 0)),
        pl.BlockSpec((1, D), lambda i: (0, 0)),
        pl.BlockSpec((NC, BR), lambda i: (0, i)),
        pl.BlockSpec((D, D), lambda i: (0, 0)),
        pl.BlockSpec((D, D), lambda i: (0, 0)),
        pl.BlockSpec((D, NCLS), lambda i: (0, 0)),
        pl.BlockSpec((1, NCLS), lambda i: (0, 0)),
        pl.BlockSpec((D, NCLS), lambda i: (0, 0)),
        pl.BlockSpec((1, NCLS), lambda i: (0, 0)),
    ],
    out_specs=[
        pl.BlockSpec((BR, D), lambda i: (i, 0)),
        pl.BlockSpec((BR, NCLS), lambda i: (i, 0)),
        pl.BlockSpec((BR, NCLS), lambda i: (i, 0)),
    ],
    out_shape=[
        jax.ShapeDtypeStruct((N, D), jnp.float32),
        jax.ShapeDtypeStruct((N, NCLS), jnp.float32),
        jax.ShapeDtypeStruct((N, NCLS), jnp.float32),
    ],
)


def _tc_mid_body(agg_ref, degp_ref, b0_ref, w1_ref, y2_ref):
    inv = _inv_sqrt_deg(degp_ref[...])
    h1 = jax.nn.relu((agg_ref[0, :, :] + agg_ref[1, :, :]) * inv + b0_ref[...])
    y2_ref[...] = jnp.dot(h1 * inv, w1_ref[...],
                          preferred_element_type=jnp.float32)


_tc_mid = pl.pallas_call(
    _tc_mid_body,
    grid=(-(-N // BR),),
    in_specs=[
        pl.BlockSpec((NC, BR, D), lambda i: (0, i, 0)),
        pl.BlockSpec((NC, BR), lambda i: (0, i)),
        pl.BlockSpec((1, D), lambda i: (0, 0)),
        pl.BlockSpec((D, D), lambda i: (0, 0)),
    ],
    out_specs=pl.BlockSpec((BR, D), lambda i: (i, 0)),
    out_shape=jax.ShapeDtypeStruct((N, D), jnp.float32),
)


def _tc_final_body(agg_ref, degp_ref, b1_ref, whc_ref, bhc_ref,
                   lut_ref, luv_ref, final_ref, lc_ref, luto_ref, luvo_ref):
    inv = _inv_sqrt_deg(degp_ref[...])
    h2 = jax.nn.relu((agg_ref[0, :, :] + agg_ref[1, :, :]) * inv + b1_ref[...])
    lc = jnp.dot(h2, whc_ref[...], preferred_element_type=jnp.float32) + bhc_ref[...]
    lc_ref[...] = lc
    lut = lut_ref[...]
    luv = luv_ref[...]
    final_ref[...] = (lc + lut + luv) / 3.0
    luto_ref[...] = lut
    luvo_ref[...] = luv


_tc_final = pl.pallas_call(
    _tc_final_body,
    grid=(-(-N // BR),),
    in_specs=[
        pl.BlockSpec((NC, BR, D), lambda i: (0, i, 0)),
        pl.BlockSpec((NC, BR), lambda i: (0, i)),
        pl.BlockSpec((1, D), lambda i: (0, 0)),
        pl.BlockSpec((D, NCLS), lambda i: (0, 0)),
        pl.BlockSpec((1, NCLS), lambda i: (0, 0)),
        pl.BlockSpec((BR, NCLS), lambda i: (i, 0)),
        pl.BlockSpec((BR, NCLS), lambda i: (i, 0)),
    ],
    out_specs=[
        pl.BlockSpec((BR, NCLS), lambda i: (i, 0)),
        pl.BlockSpec((BR, NCLS), lambda i: (i, 0)),
        pl.BlockSpec((BR, NCLS), lambda i: (i, 0)),
        pl.BlockSpec((BR, NCLS), lambda i: (i, 0)),
    ],
    out_shape=[
        jax.ShapeDtypeStruct((N, NCLS), jnp.float32),
        jax.ShapeDtypeStruct((N, NCLS), jnp.float32),
        jax.ShapeDtypeStruct((N, NCLS), jnp.float32),
        jax.ShapeDtypeStruct((N, NCLS), jnp.float32),
    ],
)


def kernel(text_feat, vis_feat, edge_index, W_t, b_t, W_v, b_v,
           W_mp0, b_mp0, W_mp1, b_mp1, W_hc, b_hc, W_ht, b_ht, W_hv, b_hv):
    pad = EPAD - E
    fake = jnp.arange(pad, dtype=jnp.int32)
    src2d = jnp.concatenate(
        [edge_index[0], fake % N]).reshape(NCH_TOT, CHUNK)
    dst2d = jnp.concatenate(
        [edge_index[1], N + fake % (NPAD - N)]).reshape(NCH_TOT, CHUNK)
    zeros1 = jnp.zeros((NPAD,), jnp.float32)
    zeros128 = jnp.zeros((NPAD, D), jnp.float32)
    ones1 = jnp.ones((CHUNK,), jnp.float32)

    degp = _sc_degree(dst2d, ones1, zeros1)
    y, lut, luv = _tc_encode(text_feat, vis_feat,
                             W_t, b_t.reshape(1, D), W_v, b_v.reshape(1, D),
                             degp, W_mp0[:D], W_mp0[D:],
                             W_ht, b_ht.reshape(1, NCLS),
                             W_hv, b_hv.reshape(1, NCLS))
    agg1 = _sc_aggregate(y, src2d, dst2d, zeros128)
    y2 = _tc_mid(agg1, degp, b_mp0.reshape(1, D), W_mp1)
    agg2 = _sc_aggregate(y2, src2d, dst2d, zeros128)
    final, lc, lut, luv = _tc_final(agg2, degp, b_mp1.reshape(1, D),
                                    W_hc, b_hc.reshape(1, NCLS), lut, luv)
    return (final, lc, lut, luv)


# speedup vs baseline: 1.0412x; 1.0412x over previous
import functools

import jax
import jax.numpy as jnp
from jax import lax
from jax.experimental import pallas as pl
from jax.experimental.pallas import tpu as pltpu
from jax.experimental.pallas import tpu_sc as plsc

N = 10000
NPAD = 10240
E = 320000
D = 128
NCLS = 40

NC = 2
NS = 16
CHUNK = 128
NCH_TOT = 2560
EPAD = NCH_TOT * CHUNK
RPT = NPAD // NS
STG = 40

_MESH = plsc.VectorSubcoreMesh(core_axis_name="c", subcore_axis_name="s")


@functools.partial(
    pl.kernel,
    out_type=jax.ShapeDtypeStruct((NC, NPAD), jnp.float32),
    mesh=_MESH,
    scratch_types=[
        pltpu.VMEM((NCH_TOT // (NC * NS), CHUNK), jnp.int32),
        pltpu.VMEM((CHUNK,), jnp.float32),
        pltpu.VMEM_SHARED((NPAD,), jnp.float32),
    ],
)
def _sc_degree(dst2d, ones1, zeros1, degp, dstv, ones_v, acc):
    c = lax.axis_index("c")
    s = lax.axis_index("s")
    n_ch = NCH_TOT // (NC * NS)

    pltpu.sync_copy(ones1, ones_v)
    pltpu.sync_copy(zeros1.at[pl.ds(s * RPT, RPT)], acc.at[pl.ds(s * RPT, RPT)])
    pltpu.sync_copy(dst2d.at[pl.ds((c * NS + s) * n_ch, n_ch)], dstv)
    plsc.subcore_barrier()

    def body(j, _):
        pltpu.sync_copy(ones_v, acc.at[dstv.at[j]], add=True)
        return 0

    lax.fori_loop(0, n_ch, body, 0)
    plsc.subcore_barrier()
    pltpu.sync_copy(acc.at[pl.ds(s * RPT, RPT)],
                    degp.at[c, pl.ds(s * RPT, RPT)])


def _gather_scatter_pipeline(table, srcv, dstv, rows, acc, gsems, ssems, n_st):
    def start_g(j):
        b = j % 2
        pltpu.async_copy(table.at[srcv.at[j]], rows.at[b], gsems.at[b])

    def wait_g(j):
        b = j % 2
        pltpu.make_async_copy(table.at[srcv.at[j]], rows.at[b],
                              gsems.at[b]).wait()

    def start_s(j):
        b = j % 2
        pltpu.async_copy(rows.at[b], acc.at[dstv.at[j]], ssems.at[b], add=True)

    def wait_s(j):
        b = j % 2
        pltpu.make_async_copy(rows.at[b], acc.at[dstv.at[j]],
                              ssems.at[b]).wait()

    start_g(0)

    def body(j, _):
        @pl.when(j >= 1)
        def _():
            wait_s(j - 1)

        @pl.when(j + 1 < n_st)
        def _():
            start_g(j + 1)

        wait_g(j)
        start_s(j)
        return 0

    lax.fori_loop(0, n_st, body, 0)
    wait_s(n_st - 1)


@functools.partial(
    pl.kernel,
    out_type=jax.ShapeDtypeStruct((NC, NPAD, D), jnp.float32),
    mesh=_MESH,
    scratch_types=[
        pltpu.VMEM((STG, CHUNK), jnp.int32),
        pltpu.VMEM((STG, CHUNK), jnp.int32),
        pltpu.VMEM((2, CHUNK, D), jnp.float32),
        pltpu.VMEM_SHARED((NPAD, D), jnp.float32),
        pltpu.SemaphoreType.DMA((2,)),
        pltpu.SemaphoreType.DMA((2,)),
    ],
)
def _sc_aggregate(y, src2d, dst2d, zeros128, agg,
                  srcv, dstv, rows, acc, gsems, ssems):
    c = lax.axis_index("c")
    s = lax.axis_index("s")
    n_ch = NCH_TOT // (NC * NS)

    pltpu.sync_copy(zeros128.at[pl.ds(s * RPT, RPT)], acc.at[pl.ds(s * RPT, RPT)])
    plsc.subcore_barrier()

    def stage(t, _):
        pltpu.sync_copy(src2d.at[pl.ds((c * NS + s) * n_ch + t * STG, STG)], srcv)
        pltpu.sync_copy(dst2d.at[pl.ds((c * NS + s) * n_ch + t * STG, STG)], dstv)
        _gather_scatter_pipeline(y, srcv, dstv, rows, acc, gsems, ssems, STG)
        return 0

    lax.fori_loop(0, n_ch // STG, stage, 0)
    plsc.subcore_barrier()
    pltpu.sync_copy(acc.at[pl.ds(s * RPT, RPT)],
                    agg.at[c, pl.ds(s * RPT, RPT)])


BR = 1024


def _inv_sqrt_deg(degp_blk):
    deg = degp_blk[0, :] + degp_blk[1, :]
    return lax.rsqrt(jnp.maximum(deg, 1.0))[:, None]


def _tc_encode_body(text_ref, vis_ref, wt_ref, bt_ref, wv_ref, bv_ref,
                    degp_ref, w0a_ref, w0b_ref, wht_ref, bht_ref,
                    whv_ref, bhv_ref, y_ref, lut_ref, luv_ref):
    h_t = jax.nn.relu(
        jnp.dot(text_ref[...], wt_ref[...], preferred_element_type=jnp.float32)
        + bt_ref[...])
    h_v = jax.nn.relu(
        jnp.dot(vis_ref[...], wv_ref[...], preferred_element_type=jnp.float32)
        + bv_ref[...])
    inv = _inv_sqrt_deg(degp_ref[...])
    y_ref[...] = (
        jnp.dot(h_t, w0a_ref[...], preferred_element_type=jnp.float32)
        + jnp.dot(h_v, w0b_ref[...], preferred_element_type=jnp.float32)) * inv
    lut_ref[...] = jnp.dot(h_t, wht_ref[...],
                           preferred_element_type=jnp.float32) + bht_ref[...]
    luv_ref[...] = jnp.dot(h_v, whv_ref[...],
                           preferred_element_type=jnp.float32) + bhv_ref[...]


_tc_encode = pl.pallas_call(
    _tc_encode_body,
    grid=(-(-N // BR),),
    in_specs=[
        pl.BlockSpec((BR, D), lambda i: (i, 0)),
        pl.BlockSpec((BR, D), lambda i: (i, 0)),
        pl.BlockSpec((D, D), lambda i: (0, 0)),
        pl.BlockSpec((1, D), lambda i: (0, 0)),
        pl.BlockSpec((D, D), lambda i: (0, 0)),
        pl.BlockSpec((1, D), lambda i: (0, 0)),
        pl.BlockSpec((NC, BR), lambda i: (0, i)),
        pl.BlockSpec((D, D), lambda i: (0, 0)),
        pl.BlockSpec((D, D), lambda i: (0, 0)),
        pl.BlockSpec((D, NCLS), lambda i: (0, 0)),
        pl.BlockSpec((1, NCLS), lambda i: (0, 0)),
        pl.BlockSpec((D, NCLS), lambda i: (0, 0)),
        pl.BlockSpec((1, NCLS), lambda i: (0, 0)),
    ],
    out_specs=[
        pl.BlockSpec((BR, D), lambda i: (i, 0)),
        pl.BlockSpec((BR, NCLS), lambda i: (i, 0)),
        pl.BlockSpec((BR, NCLS), lambda i: (i, 0)),
    ],
    out_shape=[
        jax.ShapeDtypeStruct((N, D), jnp.float32),
        jax.ShapeDtypeStruct((N, NCLS), jnp.float32),
        jax.ShapeDtypeStruct((N, NCLS), jnp.float32),
    ],
)


def _tc_mid_body(agg_ref, degp_ref, b0_ref, w1_ref, y2_ref):
    inv = _inv_sqrt_deg(degp_ref[...])
    h1 = jax.nn.relu((agg_ref[0, :, :] + agg_ref[1, :, :]) * inv + b0_ref[...])
    y2_ref[...] = jnp.dot(h1 * inv, w1_ref[...],
                          preferred_element_type=jnp.float32)


_tc_mid = pl.pallas_call(
    _tc_mid_body,
    grid=(-(-N // BR),),
    in_specs=[
        pl.BlockSpec((NC, BR, D), lambda i: (0, i, 0)),
        pl.BlockSpec((NC, BR), lambda i: (0, i)),
        pl.BlockSpec((1, D), lambda i: (0, 0)),
        pl.BlockSpec((D, D), lambda i: (0, 0)),
    ],
    out_specs=pl.BlockSpec((BR, D), lambda i: (i, 0)),
    out_shape=jax.ShapeDtypeStruct((N, D), jnp.float32),
)


def _tc_final_body(agg_ref, degp_ref, b1_ref, whc_ref, bhc_ref,
                   lut_ref, luv_ref, final_ref, lc_ref):
    inv = _inv_sqrt_deg(degp_ref[...])
    h2 = jax.nn.relu((agg_ref[0, :, :] + agg_ref[1, :, :]) * inv + b1_ref[...])
    lc = jnp.dot(h2, whc_ref[...], preferred_element_type=jnp.float32) + bhc_ref[...]
    lc_ref[...] = lc
    final_ref[...] = (lc + lut_ref[...] + luv_ref[...]) / 3.0


_tc_final = pl.pallas_call(
    _tc_final_body,
    grid=(-(-N // BR),),
    in_specs=[
        pl.BlockSpec((NC, BR, D), lambda i: (0, i, 0)),
        pl.BlockSpec((NC, BR), lambda i: (0, i)),
        pl.BlockSpec((1, D), lambda i: (0, 0)),
        pl.BlockSpec((D, NCLS), lambda i: (0, 0)),
        pl.BlockSpec((1, NCLS), lambda i: (0, 0)),
        pl.BlockSpec((BR, NCLS), lambda i: (i, 0)),
        pl.BlockSpec((BR, NCLS), lambda i: (i, 0)),
    ],
    out_specs=[
        pl.BlockSpec((BR, NCLS), lambda i: (i, 0)),
        pl.BlockSpec((BR, NCLS), lambda i: (i, 0)),
    ],
    out_shape=[
        jax.ShapeDtypeStruct((N, NCLS), jnp.float32),
        jax.ShapeDtypeStruct((N, NCLS), jnp.float32),
    ],
)


def kernel(text_feat, vis_feat, edge_index, W_t, b_t, W_v, b_v,
           W_mp0, b_mp0, W_mp1, b_mp1, W_hc, b_hc, W_ht, b_ht, W_hv, b_hv):
    pad = EPAD - E
    fake = jnp.arange(pad, dtype=jnp.int32)
    src2d = jnp.concatenate(
        [edge_index[0], fake % N]).reshape(NCH_TOT, CHUNK)
    dst2d = jnp.concatenate(
        [edge_index[1], N + fake % (NPAD - N)]).reshape(NCH_TOT, CHUNK)
    zeros1 = jnp.zeros((NPAD,), jnp.float32)
    zeros128 = jnp.zeros((NPAD, D), jnp.float32)
    ones1 = jnp.ones((CHUNK,), jnp.float32)

    degp = _sc_degree(dst2d, ones1, zeros1)
    y, lut, luv = _tc_encode(text_feat, vis_feat,
                             W_t, b_t.reshape(1, D), W_v, b_v.reshape(1, D),
                             degp, W_mp0[:D], W_mp0[D:],
                             W_ht, b_ht.reshape(1, NCLS),
                             W_hv, b_hv.reshape(1, NCLS))
    agg1 = _sc_aggregate(y, src2d, dst2d, zeros128)
    y2 = _tc_mid(agg1, degp, b_mp0.reshape(1, D), W_mp1)
    agg2 = _sc_aggregate(y2, src2d, dst2d, zeros128)
    final, lc = _tc_final(agg2, degp, b_mp1.reshape(1, D),
                          W_hc, b_hc.reshape(1, NCLS), lut, luv)
    return (final, lc, lut, luv)
